# baseline (speedup 1.0000x reference)
import functools

import jax
import jax.numpy as jnp
from jax import lax
from jax.experimental import pallas as pl
from jax.experimental.pallas import tpu as pltpu
from jax.experimental.pallas import tpu_sc as plsc

_N_FIELDS = 26
_VOCAB = 100000
_EMB = 16
_NUM_CORES = 2
_NUM_SUBCORES = 16
_LANES = 16
_HALF = 50048
_BCHUNK = 2048
_UNROLL = 4


@functools.partial(jax.jit, static_argnums=())
def _embedding_gather_t(x_t, tab_t):
    n_fields, batch = x_t.shape
    n_cols = n_fields * _EMB
    n_workers = _NUM_CORES * _NUM_SUBCORES
    cols_per_w = n_cols // n_workers
    n_chunks = batch // _BCHUNK
    groups = _BCHUNK // (_LANES * _UNROLL)
    n_stages = 2 * cols_per_w
    mesh = plsc.VectorSubcoreMesh(core_axis_name="c", subcore_axis_name="s")

    def body(x_hbm, tab_hbm, out_hbm, colA, colB, xb0, xb1, out_v,
             csemA, csemB, xsem0, xsem1):
        wid = lax.axis_index("s") * _NUM_CORES + lax.axis_index("c")
        lanes = lax.iota(jnp.int32, _LANES)
        cbufs = (colA, colB)
        csems = (csemA, csemB)
        xbufs = (xb0, xb1)
        xsems = (xsem0, xsem1)

        def col_of_stage(s):
            k, t = divmod(s, 2)
            j = wid * cols_per_w + k
            return j // _EMB, j % _EMB, t

        def start_stage(s):
            f, c, t = col_of_stage(s)
            ln = _HALF if t == 0 else _VOCAB - _HALF
            return pltpu.async_copy(
                tab_hbm.at[f, c, pl.ds(t * _HALF, ln)],
                cbufs[s % 2], csems[s % 2])

        def start_x(f, h, p):
            return pltpu.async_copy(
                x_hbm.at[f, pl.ds(h * _BCHUNK, _BCHUNK)],
                xbufs[p], xsems[p])

        def gather_chunk(xb, h, base, size, cur):
            def g16(q, carry):
                for u in range(_UNROLL):
                    o = (q * _UNROLL + u) * _LANES
                    iv = xb[pl.ds(o, _LANES)]
                    rel = iv - base
                    m = (iv >= base) & (rel < size)
                    vals = plsc.load_gather(cur, [rel], mask=m)
                    plsc.store_scatter(out_v, [lanes + (h * _BCHUNK + o)],
                                       vals, mask=m)
                return carry
            lax.fori_loop(0, groups, g16, 0)

        stage_handles = {0: start_stage(0)}
        for k in range(cols_per_w):
            j = wid * cols_per_w + k
            f = j // _EMB
            for t in range(2):
                s = 2 * k + t
                stage_handles[s].wait()
                if s + 1 < n_stages:
                    stage_handles[s + 1] = start_stage(s + 1)
                cur = cbufs[s % 2]
                base = t * _HALF
                size = _HALF if t == 0 else _VOCAB - _HALF
                xh = {0: start_x(f, 0, 0)}
                for hp in range(0, n_chunks, 2):
                    for d in range(2):
                        h = hp + d
                        if h + 1 < n_chunks:
                            xh[h + 1] = start_x(f, h + 1, (h + 1) % 2)
                        xh[h].wait()
                        gather_chunk(xbufs[h % 2], h, base, size, cur)
            pltpu.sync_copy(out_v, out_hbm.at[j, :])

    return pl.kernel(
        body,
        out_type=jax.ShapeDtypeStruct((n_cols, batch), jnp.float32),
        mesh=mesh,
        scratch_types=[
            pltpu.VMEM((_HALF,), jnp.float32),
            pltpu.VMEM((_VOCAB - _HALF,), jnp.float32),
            pltpu.VMEM((_BCHUNK,), jnp.int32),
            pltpu.VMEM((_BCHUNK,), jnp.int32),
            pltpu.VMEM((batch,), jnp.float32),
            pltpu.SemaphoreType.DMA,
            pltpu.SemaphoreType.DMA,
            pltpu.SemaphoreType.DMA,
            pltpu.SemaphoreType.DMA,
        ],
        compiler_params=pltpu.CompilerParams(use_tc_tiling_on_sc=True,
                                             needs_layout_passes=False),
    )(x_t, tab_t)


def kernel(x, tables):
    x_t = x.T
    tab_t = jnp.transpose(tables, (0, 2, 1))
    out_t = _embedding_gather_t(x_t, tab_t)
    return out_t.T


# speedup vs baseline: 1.6140x; 1.6140x over previous
import functools

import jax
import jax.numpy as jnp
from jax import lax
from jax.experimental import pallas as pl
from jax.experimental.pallas import tpu as pltpu
from jax.experimental.pallas import tpu_sc as plsc

_N_FIELDS = 26
_VOCAB = 100000
_EMB = 16
_NUM_CORES = 2
_NUM_SUBCORES = 16
_LANES = 16
_OCHUNK = 2048
_UNROLL = 8


@functools.partial(jax.jit, static_argnums=())
def _embedding_gather_t(x_t, tab_t):
    n_fields, batch = x_t.shape
    n_cols = n_fields * _EMB
    n_workers = _NUM_CORES * _NUM_SUBCORES
    cols_per_w = n_cols // n_workers
    n_blocks = batch // _OCHUNK
    groups = _OCHUNK // (_LANES * _UNROLL)
    mesh = plsc.VectorSubcoreMesh(core_axis_name="c", subcore_axis_name="s")

    def body(x_hbm, tab_hbm, out_hbm, col_v, idx_v, ob0, ob1, wsem0, wsem1):
        wid = lax.axis_index("s") * _NUM_CORES + lax.axis_index("c")
        obufs = (ob0, ob1)
        wsems = (wsem0, wsem1)

        def do_col(k, carry):
            j = wid * cols_per_w + k
            f = j // _EMB
            c = j % _EMB
            pltpu.sync_copy(tab_hbm.at[f, c, :], col_v)
            pltpu.sync_copy(x_hbm.at[f, :], idx_v)

            def do_pair(hp, prev):
                for d in range(2):
                    h = hp * 2 + d
                    ob = obufs[d]

                    def g16(q, carry2, h=h, ob=ob):
                        for u in range(_UNROLL):
                            o = (q * _UNROLL + u) * _LANES
                            iv = idx_v[pl.ds(h * _OCHUNK + o, _LANES)]
                            ob[pl.ds(o, _LANES)] = plsc.load_gather(
                                col_v, [iv])
                        return carry2

                    @pl.when(h >= 2)
                    def _(h=h, ob=ob, d=d):
                        pltpu.make_async_copy(
                            ob,
                            out_hbm.at[j, pl.ds((h - 2) * _OCHUNK, _OCHUNK)],
                            wsems[d]).wait()

                    lax.fori_loop(0, groups, g16, 0)
                    pltpu.async_copy(
                        ob, out_hbm.at[j, pl.ds(h * _OCHUNK, _OCHUNK)],
                        wsems[d])
                return prev

            lax.fori_loop(0, n_blocks // 2, do_pair, 0)
            for d in range(2):
                h = n_blocks - 2 + d
                pltpu.make_async_copy(
                    obufs[d],
                    out_hbm.at[j, pl.ds(h * _OCHUNK, _OCHUNK)],
                    wsems[d]).wait()
            return carry

        lax.fori_loop(0, cols_per_w, do_col, 0)

    return pl.kernel(
        body,
        out_type=jax.ShapeDtypeStruct((n_cols, batch), jnp.float32),
        mesh=mesh,
        scratch_types=[
            pltpu.VMEM((_VOCAB,), jnp.float32),
            pltpu.VMEM((batch,), jnp.int32),
            pltpu.VMEM((_OCHUNK,), jnp.float32),
            pltpu.VMEM((_OCHUNK,), jnp.float32),
            pltpu.SemaphoreType.DMA,
            pltpu.SemaphoreType.DMA,
        ],
        compiler_params=pltpu.CompilerParams(use_tc_tiling_on_sc=True,
                                             needs_layout_passes=False),
    )(x_t, tab_t)


def kernel(x, tables):
    x_t = x.T
    tab_t = jnp.transpose(tables, (0, 2, 1))
    out_t = _embedding_gather_t(x_t, tab_t)
    return out_t.T


# speedup vs baseline: 2.9374x; 1.8200x over previous
import functools

import jax
import jax.numpy as jnp
from jax import lax
from jax.experimental import pallas as pl
from jax.experimental.pallas import tpu as pltpu
from jax.experimental.pallas import tpu_sc as plsc

_N_FIELDS = 26
_VOCAB = 100000
_EMB = 16
_NUM_CORES = 2
_NUM_SUBCORES = 16
_LANES = 16
_OCHUNK = 2048
_UNROLL = 8


@functools.partial(jax.jit, static_argnums=())
def _embedding_gather_t(x_t, tab_t):
    n_fields, batch = x_t.shape
    n_cols = n_fields * _EMB
    n_workers = _NUM_CORES * _NUM_SUBCORES
    cols_per_w = n_cols // n_workers
    n_blocks = batch // _OCHUNK
    groups = _OCHUNK // (_LANES * _UNROLL)
    mesh = plsc.VectorSubcoreMesh(core_axis_name="c", subcore_axis_name="s")

    def body(x_hbm, tab_hbm, out_hbm, col_v, idx_v, ob0, ob1, wsem0, wsem1):
        wid = lax.axis_index("s") * _NUM_CORES + lax.axis_index("c")
        obufs = (ob0, ob1)
        wsems = (wsem0, wsem1)

        def do_col(k, carry):
            j = wid * cols_per_w + k
            f = j // _EMB
            c = j % _EMB
            pltpu.sync_copy(tab_hbm.at[f, c, :], col_v)
            pltpu.sync_copy(x_hbm.at[f, :], idx_v)

            def do_pair(hp, prev):
                for d in range(2):
                    h = hp * 2 + d
                    ob = obufs[d]

                    @pl.when(h >= 2)
                    def _(h=h, ob=ob, d=d):
                        pltpu.make_async_copy(
                            ob,
                            out_hbm.at[j, pl.ds((h - 2) * _OCHUNK, _OCHUNK)],
                            wsems[d]).wait()

                    @plsc.parallel_loop(0, _OCHUNK, step=_LANES,
                                        unroll=_UNROLL)
                    def _(o, h=h, ob=ob):
                        iv = idx_v[pl.ds(h * _OCHUNK + o, _LANES)]
                        ob[pl.ds(o, _LANES)] = plsc.load_gather(col_v, [iv])
                    pltpu.async_copy(
                        ob, out_hbm.at[j, pl.ds(h * _OCHUNK, _OCHUNK)],
                        wsems[d])
                return prev

            lax.fori_loop(0, n_blocks // 2, do_pair, 0)
            for d in range(2):
                h = n_blocks - 2 + d
                pltpu.make_async_copy(
                    obufs[d],
                    out_hbm.at[j, pl.ds(h * _OCHUNK, _OCHUNK)],
                    wsems[d]).wait()
            return carry

        lax.fori_loop(0, cols_per_w, do_col, 0)

    return pl.kernel(
        body,
        out_type=jax.ShapeDtypeStruct((n_cols, batch), jnp.float32),
        mesh=mesh,
        scratch_types=[
            pltpu.VMEM((_VOCAB,), jnp.float32),
            pltpu.VMEM((batch,), jnp.int32),
            pltpu.VMEM((_OCHUNK,), jnp.float32),
            pltpu.VMEM((_OCHUNK,), jnp.float32),
            pltpu.SemaphoreType.DMA,
            pltpu.SemaphoreType.DMA,
        ],
        compiler_params=pltpu.CompilerParams(use_tc_tiling_on_sc=True,
                                             needs_layout_passes=False),
    )(x_t, tab_t)


def kernel(x, tables):
    x_t = x.T
    tab_t = jnp.transpose(tables, (0, 2, 1))
    out_t = _embedding_gather_t(x_t, tab_t)
    return out_t.T
